# baseline (device time: 288404 ns/iter reference)
import jax
import jax.numpy as jnp
from jax import lax
from jax.experimental import pallas as pl
from jax.experimental.pallas import tpu as pltpu

N_DEV = 8
M = 1536
D = 1536
H = 3072
BH = 384
R = M // N_DEV


def _mlp_body(x_ref, wg_ref, wu_ref, wd_ref, out_ref, x16_ref):
    k = pl.program_id(0)

    @pl.when(k == 0)
    def _():
        x16_ref[...] = x_ref[...].astype(jnp.bfloat16)

    x16 = x16_ref[...]
    g = jnp.dot(x16, wg_ref[...].astype(jnp.bfloat16),
                preferred_element_type=jnp.float32)
    u = jnp.dot(x16, wu_ref[...].astype(jnp.bfloat16),
                preferred_element_type=jnp.float32)
    a = g * (u * jax.nn.sigmoid(u))
    part = jnp.dot(a.astype(jnp.bfloat16), wd_ref[...].astype(jnp.bfloat16),
                   preferred_element_type=jnp.float32)

    @pl.when(k == 0)
    def _():
        out_ref[...] = part

    @pl.when(k > 0)
    def _():
        out_ref[...] += part


def _mlp(x, wg, wu, wd):
    return pl.pallas_call(
        _mlp_body,
        grid=(H // BH,),
        in_specs=[
            pl.BlockSpec((M, D), lambda k: (0, 0)),
            pl.BlockSpec((D, BH), lambda k: (0, k)),
            pl.BlockSpec((D, BH), lambda k: (0, k)),
            pl.BlockSpec((BH, D), lambda k: (k, 0)),
        ],
        out_specs=pl.BlockSpec((M, D), lambda k: (0, 0)),
        out_shape=jax.ShapeDtypeStruct((M, D), jnp.float32),
        scratch_shapes=[pltpu.VMEM((M, D), jnp.bfloat16)],
    )(x, wg, wu, wd)


def _ar_body(p_ref, out_ref, rs_buf,
             rs_send_sems, rs_recv_sems, ag_send_sems, ag_recv_sems):
    i = lax.axis_index("i")
    left = lax.rem(i - 1 + N_DEV, N_DEV)
    right = lax.rem(i + 1, N_DEV)

    barrier = pltpu.get_barrier_semaphore()
    for nbr in (left, right):
        pl.semaphore_signal(barrier, inc=1, device_id=(nbr,),
                            device_id_type=pl.DeviceIdType.MESH)
    pl.semaphore_wait(barrier, 2)

    for s in range(N_DEV - 1):
        c_send = lax.rem(i - s + N_DEV, N_DEV)
        if s == 0:
            src = p_ref.at[pl.ds(c_send * R, R), :]
        else:
            src = rs_buf.at[s - 1]
        rdma = pltpu.make_async_remote_copy(
            src_ref=src,
            dst_ref=rs_buf.at[s],
            send_sem=rs_send_sems.at[s],
            recv_sem=rs_recv_sems.at[s],
            device_id=(right,),
            device_id_type=pl.DeviceIdType.MESH,
        )
        rdma.start()
        rdma.wait()
        c_recv = lax.rem(i - s - 1 + N_DEV, N_DEV)
        rs_buf[s, :, :] = rs_buf[s, :, :] + p_ref[pl.ds(c_recv * R, R), :]

    mine = lax.rem(i + 1, N_DEV)
    out_ref[pl.ds(mine * R, R), :] = rs_buf[N_DEV - 2, :, :]

    for t in range(N_DEV - 1):
        c = lax.rem(i + 1 - t + N_DEV, N_DEV)
        rdma = pltpu.make_async_remote_copy(
            src_ref=out_ref.at[pl.ds(c * R, R), :],
            dst_ref=out_ref.at[pl.ds(c * R, R), :],
            send_sem=ag_send_sems.at[t],
            recv_sem=ag_recv_sems.at[t],
            device_id=(right,),
            device_id_type=pl.DeviceIdType.MESH,
        )
        rdma.start()
        rdma.wait()


def _all_reduce(partial):
    return pl.pallas_call(
        _ar_body,
        out_shape=jax.ShapeDtypeStruct((M, D), jnp.float32),
        in_specs=[pl.BlockSpec(memory_space=pltpu.VMEM)],
        out_specs=pl.BlockSpec(memory_space=pltpu.VMEM),
        scratch_shapes=[
            pltpu.VMEM((N_DEV - 1, R, D), jnp.float32),
            pltpu.SemaphoreType.DMA((N_DEV - 1,)),
            pltpu.SemaphoreType.DMA((N_DEV - 1,)),
            pltpu.SemaphoreType.DMA((N_DEV - 1,)),
            pltpu.SemaphoreType.DMA((N_DEV - 1,)),
        ],
        compiler_params=pltpu.CompilerParams(collective_id=0),
    )(partial)


def kernel(x, Wg, Wu, Wd):
    partial = _mlp(x, Wg, Wu, Wd)
    return _all_reduce(partial)


# device time: 161291 ns/iter; 1.7881x vs baseline; 1.7881x over previous
import jax
import jax.numpy as jnp
from jax import lax
from jax.experimental import pallas as pl
from jax.experimental.pallas import tpu as pltpu

N_DEV = 8
M = 1536
D = 1536
H = 3072
BH = 384
R = M // N_DEV


def _mlp_body(x_ref, wg_ref, wu_ref, wd_ref, out_ref, x16_ref):
    k = pl.program_id(0)

    @pl.when(k == 0)
    def _():
        x16_ref[...] = x_ref[...].astype(jnp.bfloat16)

    x16 = x16_ref[...]
    g = jnp.dot(x16, wg_ref[...].astype(jnp.bfloat16),
                preferred_element_type=jnp.float32)
    u = jnp.dot(x16, wu_ref[...].astype(jnp.bfloat16),
                preferred_element_type=jnp.float32)
    a = g * (u * jax.nn.sigmoid(u))
    part = jnp.dot(a.astype(jnp.bfloat16), wd_ref[...].astype(jnp.bfloat16),
                   preferred_element_type=jnp.float32)

    @pl.when(k == 0)
    def _():
        out_ref[...] = part

    @pl.when(k > 0)
    def _():
        out_ref[...] += part


def _mlp(x, wg, wu, wd):
    return pl.pallas_call(
        _mlp_body,
        grid=(H // BH,),
        in_specs=[
            pl.BlockSpec((M, D), lambda k: (0, 0)),
            pl.BlockSpec((D, BH), lambda k: (0, k)),
            pl.BlockSpec((D, BH), lambda k: (0, k)),
            pl.BlockSpec((BH, D), lambda k: (k, 0)),
        ],
        out_specs=pl.BlockSpec((M, D), lambda k: (0, 0)),
        out_shape=jax.ShapeDtypeStruct((M, D), jnp.float32),
        scratch_shapes=[pltpu.VMEM((M, D), jnp.bfloat16)],
    )(x, wg, wu, wd)


COLS = D // 2


def _ar_body(p_ref, out_ref,
             rs_buf_a, rs_buf_b, send_buf_a, send_buf_b,
             ag_buf_a, ag_buf_b, init_a, init_b,
             rs_send_a, rs_recv_a, rs_send_b, rs_recv_b,
             ag_send_a, ag_recv_a, ag_send_b, ag_recv_b):
    i = lax.axis_index("i")
    left = lax.rem(i - 1 + N_DEV, N_DEV)
    right = lax.rem(i + 1, N_DEV)

    barrier = pltpu.get_barrier_semaphore()
    for nbr in (left, right):
        pl.semaphore_signal(barrier, inc=1, device_id=(nbr,),
                            device_id_type=pl.DeviceIdType.MESH)
    pl.semaphore_wait(barrier, 2)

    row0 = i * R
    init_a[...] = p_ref[pl.ds(row0, R), pl.ds(0, COLS)].astype(jnp.bfloat16)
    init_b[...] = p_ref[pl.ds(row0, R), pl.ds(COLS, COLS)].astype(jnp.bfloat16)

    for s in range(N_DEV - 1):
        rd_a = pltpu.make_async_remote_copy(
            src_ref=init_a if s == 0 else send_buf_a.at[s - 1],
            dst_ref=rs_buf_a.at[s],
            send_sem=rs_send_a.at[s],
            recv_sem=rs_recv_a.at[s],
            device_id=(right,),
            device_id_type=pl.DeviceIdType.MESH,
        )
        rd_b = pltpu.make_async_remote_copy(
            src_ref=init_b if s == 0 else send_buf_b.at[s - 1],
            dst_ref=rs_buf_b.at[s],
            send_sem=rs_send_b.at[s],
            recv_sem=rs_recv_b.at[s],
            device_id=(left,),
            device_id_type=pl.DeviceIdType.MESH,
        )
        rd_a.start()
        rd_b.start()
        rd_a.wait()
        rd_b.wait()
        ca = lax.rem(i - s - 1 + N_DEV, N_DEV)
        cb = lax.rem(i + s + 1, N_DEV)
        acc_a = (rs_buf_a[s, :, :].astype(jnp.float32)
                 + p_ref[pl.ds(ca * R, R), pl.ds(0, COLS)])
        acc_b = (rs_buf_b[s, :, :].astype(jnp.float32)
                 + p_ref[pl.ds(cb * R, R), pl.ds(COLS, COLS)])
        send_buf_a[s, :, :] = acc_a.astype(jnp.bfloat16)
        send_buf_b[s, :, :] = acc_b.astype(jnp.bfloat16)
        if s == N_DEV - 2:
            out_ref[pl.ds(ca * R, R), pl.ds(0, COLS)] = acc_a
            out_ref[pl.ds(cb * R, R), pl.ds(COLS, COLS)] = acc_b

    def _ag_start(t):
        rd_a = pltpu.make_async_remote_copy(
            src_ref=send_buf_a.at[N_DEV - 2] if t == 0 else ag_buf_a.at[t - 1],
            dst_ref=ag_buf_a.at[t],
            send_sem=ag_send_a.at[t],
            recv_sem=ag_recv_a.at[t],
            device_id=(right,),
            device_id_type=pl.DeviceIdType.MESH,
        )
        rd_b = pltpu.make_async_remote_copy(
            src_ref=send_buf_b.at[N_DEV - 2] if t == 0 else ag_buf_b.at[t - 1],
            dst_ref=ag_buf_b.at[t],
            send_sem=ag_send_b.at[t],
            recv_sem=ag_recv_b.at[t],
            device_id=(left,),
            device_id_type=pl.DeviceIdType.MESH,
        )
        rd_a.start()
        rd_b.start()
        return rd_a, rd_b

    pending = _ag_start(0)
    for t in range(N_DEV - 1):
        rd_a, rd_b = pending
        rd_a.wait()
        rd_b.wait()
        if t < N_DEV - 2:
            pending = _ag_start(t + 1)
        ca = lax.rem(i - t + N_DEV, N_DEV)
        cb = lax.rem(i + t, N_DEV)
        out_ref[pl.ds(ca * R, R), pl.ds(0, COLS)] = (
            ag_buf_a[t, :, :].astype(jnp.float32))
        out_ref[pl.ds(cb * R, R), pl.ds(COLS, COLS)] = (
            ag_buf_b[t, :, :].astype(jnp.float32))


def _all_reduce(partial):
    n_steps = N_DEV - 1
    buf = pltpu.VMEM((n_steps, R, COLS), jnp.bfloat16)
    sems = pltpu.SemaphoreType.DMA((n_steps,))
    return pl.pallas_call(
        _ar_body,
        out_shape=jax.ShapeDtypeStruct((M, D), jnp.float32),
        in_specs=[pl.BlockSpec(memory_space=pltpu.VMEM)],
        out_specs=pl.BlockSpec(memory_space=pltpu.VMEM),
        scratch_shapes=(
            [buf] * 6
            + [pltpu.VMEM((R, COLS), jnp.bfloat16)] * 2
            + [sems] * 8
        ),
        compiler_params=pltpu.CompilerParams(collective_id=0),
    )(partial)


def kernel(x, Wg, Wu, Wd):
    partial = _mlp(x, Wg, Wu, Wd)
    return _all_reduce(partial)
